# baseline (speedup 1.0000x reference)
import dataclasses
import functools

import jax
import jax.numpy as jnp
from jax import lax
from jax.experimental import pallas as pl
from jax.experimental.pallas import tpu as pltpu
from jax.experimental.pallas import tpu_sc as plsc

B = 4096
L = 200
D = 64
V = 1000001
DH = D // 2
NC = 2
NS = 16
NW = NC * NS
BPW = B // NW
CH = 100
NCH = L // CH
LANES = 16
WPR = DH // 2

_TCB = 4096
_NBLK = 31
Q8 = _NBLK * _TCB
VP = 8 * Q8
_LASTB = (V - 1) // _TCB

_mesh = plsc.VectorSubcoreMesh(core_axis_name="c", subcore_axis_name="s")

_sc_params = dataclasses.replace(
    pltpu.CompilerParams(use_tc_tiling_on_sc=False),
    **(
        {"needs_layout_passes": False}
        if "needs_layout_passes" in pltpu.CompilerParams.__dataclass_fields__
        else {}
    ),
)


@functools.partial(
    pl.kernel,
    mesh=_mesh,
    out_type=jax.ShapeDtypeStruct((B, DH), jnp.float32),
    compiler_params=_sc_params,
    scratch_types=[
        pltpu.VMEM((BPW * NCH, CH), jnp.int32),
        pltpu.VMEM((2, NCH, CH, WPR), jnp.int32),
        pltpu.VMEM((BPW, DH), jnp.float32),
        pltpu.SemaphoreType.DMA((2,)),
    ],
)
def _emb_pool(idx_hbm, tab_hbm, out_hbm, idx_v, gbuf, out_v, sem):
    wid = lax.axis_index("s") * NC + lax.axis_index("c")
    row0 = wid * (BPW * NCH)
    pltpu.sync_copy(idx_hbm.at[pl.ds(row0, BPW * NCH)], idx_v)

    def issue(s, b):
        for c in range(NCH):
            pltpu.make_async_copy(
                tab_hbm.at[idx_v.at[s * NCH + c]],
                gbuf.at[b, c],
                sem.at[b],
            ).start()

    def wait(b):
        for c in range(NCH):
            pltpu.make_async_copy(
                tab_hbm.at[idx_v.at[c]],
                gbuf.at[b, c],
                sem.at[b],
            ).wait()

    def accum_store(s, b):
        zero = jnp.zeros((LANES,), jnp.float32)
        acc = [zero, zero]

        def row(j, acc):
            w = gbuf[b, c, j, pl.ds(0, LANES)]
            lo, hi = plsc.unpack(
                plsc.bitcast(w, jnp.bfloat16),
                format=plsc.PackFormat.INTERLEAVED,
            )
            return [acc[0] + lo, acc[1] + hi]

        for c in range(NCH):
            def body4(j4, acc, c=c):
                for r in range(4):
                    acc = row(j4 * 4 + r, acc)
                return acc

            acc = lax.fori_loop(0, CH // 4, body4, acc)

        out_v[s, pl.ds(0, LANES)] = acc[0]
        out_v[s, pl.ds(LANES, LANES)] = acc[1]

    issue(0, 0)

    @pl.loop(0, BPW, step=2)
    def _(s):
        issue(s + 1, 1)
        wait(0)
        accum_store(s, 0)

        @pl.when(s + 2 < BPW)
        def _():
            issue(s + 2, 0)

        wait(1)
        accum_store(s + 1, 1)

    pltpu.sync_copy(out_v, out_hbm.at[pl.ds(wid * BPW, BPW)])


def _pack_bf16(t):
    u = lax.bitcast_convert_type(t, jnp.uint32) + 0x8000
    w = (u[:WPR, :] >> 16) | (u[WPR:, :] & jnp.uint32(0xFFFF0000))
    return lax.bitcast_convert_type(w, jnp.int32)


def _tc_transpose_pack(tT, h):
    def body(i0, i1, i2, i3, i4, i5, i6, i7, out_ref):
        out_ref[...] = jnp.concatenate(
            [_pack_bf16(r[...]).T for r in (i0, i1, i2, i3, i4, i5, i6, i7)],
            axis=1,
        )

    specs = [
        pl.BlockSpec(
            (DH, _TCB),
            lambda j, q=q: (h, jnp.minimum(j + q * _NBLK, _LASTB)),
        )
        for q in range(8)
    ]
    return pl.pallas_call(
        body,
        grid=(_NBLK,),
        in_specs=specs,
        out_specs=pl.BlockSpec((_TCB, 8 * WPR), lambda j: (j, 0)),
        out_shape=jax.ShapeDtypeStruct((Q8, 8 * WPR), jnp.int32),
    )(*([tT] * 8))


def kernel(indices, table):
    idx = indices.astype(jnp.int32)
    idxr = (idx % Q8) * 8 + idx // Q8
    idx2 = idxr.reshape(B * L // CH, CH)
    tT = jnp.swapaxes(table, 0, 1)
    outs = []
    for h in range(2):
        tab = _tc_transpose_pack(tT, h).reshape(VP, WPR)
        outs.append(_emb_pool(idx2, tab))
    return jnp.concatenate(outs, axis=1)


# speedup vs baseline: 1.7844x; 1.7844x over previous
import dataclasses
import functools

import jax
import jax.numpy as jnp
from jax import lax
from jax.experimental import pallas as pl
from jax.experimental.pallas import tpu as pltpu
from jax.experimental.pallas import tpu_sc as plsc

B = 4096
L = 200
D = 64
V = 1000001
NC = 2
NS = 16
NW = NC * NS
BPW = B // NW
CH = 100
NCH = L // CH
LANES = 16
WPR = D // 2
NVR = D // LANES

_TCB = 8192
_NBLK = 31
Q = _NBLK * _TCB
VP = 4 * Q
_LASTB = (V - 1) // _TCB

_mesh = plsc.VectorSubcoreMesh(core_axis_name="c", subcore_axis_name="s")


@functools.partial(
    pl.kernel,
    mesh=_mesh,
    out_type=jax.ShapeDtypeStruct((B, D), jnp.float32),
    compiler_params=dataclasses.replace(
        pltpu.CompilerParams(use_tc_tiling_on_sc=False),
        **(
            {"needs_layout_passes": False}
            if "needs_layout_passes" in pltpu.CompilerParams.__dataclass_fields__
            else {}
        ),
    ),
    scratch_types=[
        pltpu.VMEM((BPW * NCH, CH), jnp.int32),
        pltpu.VMEM((2, NCH, CH, WPR), jnp.int32),
        pltpu.VMEM((BPW, D), jnp.float32),
        pltpu.SemaphoreType.DMA((2,)),
    ],
)
def _emb_pool(idx_hbm, tab_hbm, out_hbm, idx_v, gbuf, out_v, sem):
    wid = lax.axis_index("s") * NC + lax.axis_index("c")
    row0 = wid * (BPW * NCH)
    pltpu.sync_copy(idx_hbm.at[pl.ds(row0, BPW * NCH)], idx_v)

    def issue(s, b):
        for c in range(NCH):
            pltpu.make_async_copy(
                tab_hbm.at[idx_v.at[s * NCH + c]],
                gbuf.at[b, c],
                sem.at[b],
            ).start()

    def wait(b):
        for c in range(NCH):
            pltpu.make_async_copy(
                tab_hbm.at[idx_v.at[c]],
                gbuf.at[b, c],
                sem.at[b],
            ).wait()

    def accum_store(s, b):
        zero = jnp.zeros((LANES,), jnp.float32)
        acc = [zero] * NVR

        def row(j, acc):
            out = list(acc)
            for k in range(2):
                w = gbuf[b, c, j, pl.ds(k * LANES, LANES)]
                lo, hi = plsc.unpack(
                    plsc.bitcast(w, jnp.bfloat16),
                    format=plsc.PackFormat.INTERLEAVED,
                )
                out[k] = out[k] + lo
                out[k + 2] = out[k + 2] + hi
            return out

        for c in range(NCH):
            def body4(j4, acc, c=c):
                for r in range(4):
                    acc = row(j4 * 4 + r, acc)
                return acc

            acc = lax.fori_loop(0, CH // 4, body4, acc)

        for k in range(NVR):
            out_v[s, pl.ds(k * LANES, LANES)] = acc[k]

    issue(0, 0)

    @pl.loop(0, BPW, step=2)
    def _(s):
        issue(s + 1, 1)
        wait(0)
        accum_store(s, 0)

        @pl.when(s + 2 < BPW)
        def _():
            issue(s + 2, 0)

        wait(1)
        accum_store(s + 1, 1)

    pltpu.sync_copy(out_v, out_hbm.at[pl.ds(wid * BPW, BPW)])


def _pack_bf16(t):
    u = lax.bitcast_convert_type(t, jnp.uint32) + 0x8000
    w = (u[:WPR, :] >> 16) | (u[WPR:, :] & jnp.uint32(0xFFFF0000))
    return lax.bitcast_convert_type(w, jnp.int32)


def _tc_transpose_pack(tT):
    def body(i0, i1, i2, i3, out_ref):
        out_ref[...] = jnp.concatenate(
            [_pack_bf16(r[...]).T for r in (i0, i1, i2, i3)], axis=1
        )

    specs = [
        pl.BlockSpec((D, _TCB), lambda j, q=q: (0, jnp.minimum(j + q * _NBLK, _LASTB)))
        for q in range(4)
    ]
    return pl.pallas_call(
        body,
        grid=(_NBLK,),
        in_specs=specs,
        out_specs=pl.BlockSpec((_TCB, 2 * D), lambda j: (j, 0)),
        out_shape=jax.ShapeDtypeStruct((Q, 2 * D), jnp.int32),
    )(tT, tT, tT, tT)


def kernel(indices, table):
    idx = indices.astype(jnp.int32)
    idxr = (idx % Q) * 4 + idx // Q
    idx2 = idxr.reshape(B * L // CH, CH)
    tab = _tc_transpose_pack(jnp.swapaxes(table, 0, 1)).reshape(VP, WPR)
    return _emb_pool(idx2, tab)
